# baseline (speedup 1.0000x reference)
import math

import jax
import jax.numpy as jnp
from jax import lax
from jax.experimental import pallas as pl
from jax.experimental.pallas import tpu as pltpu

_B, _N, _FEAT = 64, 36, 2048
_K = 16
_NK = 8
_MID = 1024
_BM = 16
_BMN = _BM * _N


def _graph_conv_body(c2_ref, adj_ref, ti_ref, v_ref, wt_ref,
                     mr_ref, mt_ref, pr_ref, pt_ref, out_ref,
                     wv_ref, t_ref):
    n, nk, k = _N, _NK, _K

    wcat = wt_ref[...].reshape(_MID, _FEAT)
    vall = v_ref[...].reshape(_BMN, _FEAT)
    wv_ref[...] = lax.dot_general(vall, wcat,
                                  dimension_numbers=(((1,), (1,)), ((), ())),
                                  preferred_element_type=jnp.float32)

    r_iota = lax.broadcasted_iota(jnp.int32, (2 * n, n), 0)
    c_iota = lax.broadcasted_iota(jnp.int32, (2 * n, n), 1)
    s_rho = (r_iota == 2 * c_iota).astype(jnp.float32)
    s_theta = (r_iota == 2 * c_iota + 1).astype(jnp.float32)
    c_all = c2_ref[...].reshape(_BMN, 2 * n)
    rho = jnp.dot(c_all, s_rho, preferred_element_type=jnp.float32)
    theta = jnp.dot(c_all, s_theta, preferred_element_type=jnp.float32)

    lane = lax.broadcasted_iota(jnp.int32, (_BMN, n), 1)
    ti_all = ti_ref[...].reshape(_BMN, k)
    adj_all = adj_ref[...].reshape(_BMN, k)
    acc = jnp.zeros((_BMN, n), dtype=jnp.float32)
    for kk in range(k):
        acc = acc + jnp.where(lane == ti_all[:, kk:kk + 1],
                              adj_all[:, kk:kk + 1], 0.0)

    ws = []
    for i in range(nk):
        d = (rho - mr_ref[i, 0]) ** 2
        w_r = jnp.exp(-0.5 * d / (1e-14 + pr_ref[i, 0] ** 2))
        fa = jnp.abs(theta - mt_ref[i, 0])
        sa = jnp.abs(2.0 * math.pi - fa)
        ang = jnp.minimum(fa, sa)
        w_t = jnp.exp(-0.5 * ang * ang / (1e-14 + pt_ref[i, 0] ** 2))
        w = w_r * w_t
        w = jnp.where(jnp.isnan(w), 0.0, w)
        ws.append(w)
    wsum = ws[0]
    for i in range(1, nk):
        wsum = wsum + ws[i]
    scaled = acc / (wsum + 1e-14)
    for i in range(nk):
        t_ref[i] = ws[i] * scaled

    for b in range(_BM):
        base = b * n
        for i in range(nk):
            out_ref[b, :, i * 128:(i + 1) * 128] = jnp.dot(
                t_ref[i, base:base + n, :],
                wv_ref[base:base + n, i * 128:(i + 1) * 128],
                preferred_element_type=jnp.float32)


@jax.jit
def _graph_conv(v, c2, adj, ti, W, mr, mt, pr, pt):
    grid = _B // _BM
    out = pl.pallas_call(
        _graph_conv_body,
        grid=(grid,),
        in_specs=[
            pl.BlockSpec((_BM, _N, 2 * _N), lambda i: (i, 0, 0)),
            pl.BlockSpec((_BM, _N, _K), lambda i: (i, 0, 0)),
            pl.BlockSpec((_BM, _N, _K), lambda i: (i, 0, 0)),
            pl.BlockSpec((_BM, _N, _FEAT), lambda i: (i, 0, 0)),
            pl.BlockSpec((_NK, 128, _FEAT), lambda i: (0, 0, 0)),
            pl.BlockSpec((_NK, 1), lambda i: (0, 0)),
            pl.BlockSpec((_NK, 1), lambda i: (0, 0)),
            pl.BlockSpec((_NK, 1), lambda i: (0, 0)),
            pl.BlockSpec((_NK, 1), lambda i: (0, 0)),
        ],
        out_specs=pl.BlockSpec((_BM, _N, _MID), lambda i: (i, 0, 0)),
        out_shape=jax.ShapeDtypeStruct((_B, _N, _MID), jnp.float32),
        scratch_shapes=[pltpu.VMEM((_BMN, _MID), jnp.float32),
                        pltpu.VMEM((_NK, _BMN, _N), jnp.float32)],
    )(c2, adj, ti, v, W, mr, mt, pr, pt)
    return out


def kernel(v, v_mask, coord, adj_matrix, top_ind, W, mean_rho, mean_theta,
           precision_rho, precision_theta):
    del v_mask
    c2 = coord.reshape(_B, _N, 2 * _N)
    return _graph_conv(v, c2, adj_matrix, top_ind, W, mean_rho, mean_theta,
                       precision_rho, precision_theta)


# speedup vs baseline: 1.0589x; 1.0589x over previous
import math

import jax
import jax.numpy as jnp
from jax import lax
from jax.experimental import pallas as pl
from jax.experimental.pallas import tpu as pltpu

_B, _N, _FEAT = 64, 36, 2048
_K = 16
_NK = 8
_MID = 1024
_BM = 16
_BMN = _BM * _N
_PK = 512


def _graph_conv_body(pk_ref, v_ref, wt_ref, pp_ref, out_ref, wv_ref):
    n, nk, k = _N, _NK, _K

    wcat = wt_ref[...].reshape(_MID, _FEAT)
    vall = v_ref[...].reshape(_BMN, _FEAT)
    wv_ref[...] = lax.dot_general(vall, wcat,
                                  dimension_numbers=(((1,), (1,)), ((), ())),
                                  preferred_element_type=jnp.float32)

    iota_j = lax.broadcasted_iota(jnp.int32, (n, n), 1)

    for b in range(_BM):
        rho = pk_ref[b, :, 0:n]
        theta = pk_ref[b, :, 128:128 + n]
        adj = pk_ref[b, :, 256:256 + k]
        tif = pk_ref[b, :, 384:384 + k]

        ws = []
        for i in range(nk):
            d = (rho - pp_ref[i, 0]) ** 2
            w_r = jnp.exp(-0.5 * d / (1e-14 + pp_ref[i, 2] ** 2))
            fa = jnp.abs(theta - pp_ref[i, 1])
            sa = jnp.abs(2.0 * math.pi - fa)
            ang = jnp.minimum(fa, sa)
            w_t = jnp.exp(-0.5 * ang * ang / (1e-14 + pp_ref[i, 3] ** 2))
            w = w_r * w_t
            w = jnp.where(jnp.isnan(w), 0.0, w)
            ws.append(w)
        wsum = ws[0]
        for i in range(1, nk):
            wsum = wsum + ws[i]
        inv = 1.0 / (wsum + 1e-14)

        acc = jnp.zeros((n, n), dtype=jnp.float32)
        fiota = iota_j.astype(jnp.float32)
        for kk in range(k):
            idx = tif[:, kk:kk + 1]
            val = adj[:, kk:kk + 1]
            acc = acc + jnp.where(fiota == idx, val, 0.0)
        scaled = acc * inv

        base = b * n
        wv_b = wv_ref[base:base + n, :]
        for i in range(nk):
            mi = ws[i] * scaled
            out_ref[b, :, i * 128:(i + 1) * 128] = jnp.dot(
                mi, wv_b[:, i * 128:(i + 1) * 128],
                preferred_element_type=jnp.float32)


@jax.jit
def _graph_conv(v, pk, W, pp):
    grid = _B // _BM
    out = pl.pallas_call(
        _graph_conv_body,
        grid=(grid,),
        in_specs=[
            pl.BlockSpec((_BM, _N, _PK), lambda i: (i, 0, 0)),
            pl.BlockSpec((_BM, _N, _FEAT), lambda i: (i, 0, 0)),
            pl.BlockSpec((_NK, 128, _FEAT), lambda i: (0, 0, 0)),
            pl.BlockSpec((_NK, 4), lambda i: (0, 0)),
        ],
        out_specs=pl.BlockSpec((_BM, _N, _MID), lambda i: (i, 0, 0)),
        out_shape=jax.ShapeDtypeStruct((_B, _N, _MID), jnp.float32),
        scratch_shapes=[pltpu.VMEM((_BMN, _MID), jnp.float32)],
    )(pk, v, W, pp)
    return out


def kernel(v, v_mask, coord, adj_matrix, top_ind, W, mean_rho, mean_theta,
           precision_rho, precision_theta):
    del v_mask
    z92 = jnp.zeros((_B, _N, 128 - _N), jnp.float32)
    z112 = jnp.zeros((_B, _N, 128 - _K), jnp.float32)
    pk = jnp.concatenate(
        [coord[:, :, :, 0], z92, coord[:, :, :, 1], z92, adj_matrix, z112,
         top_ind.astype(jnp.float32), z112], axis=-1)
    pp = jnp.concatenate(
        [mean_rho, mean_theta, precision_rho, precision_theta], axis=-1)
    return _graph_conv(v, pk, W, pp)


# speedup vs baseline: 1.1220x; 1.0596x over previous
import math

import jax
import jax.numpy as jnp
from jax import lax
from jax.experimental import pallas as pl
from jax.experimental.pallas import tpu as pltpu

_B, _N, _FEAT = 64, 36, 2048
_K = 16
_NK = 8
_MID = 1024
_BM = 16
_BMN = _BM * _N
_PK = 256


def _graph_conv_body(pk_ref, adj_ref, ti_ref, v_ref, wt_ref, pp_ref, out_ref, wv_ref):
    n, nk, k = _N, _NK, _K

    wcat = wt_ref[...].reshape(_MID, _FEAT)
    vall = v_ref[...].reshape(_BMN, _FEAT)
    wv_ref[...] = lax.dot_general(vall, wcat,
                                  dimension_numbers=(((1,), (1,)), ((), ())),
                                  preferred_element_type=jnp.float32)

    iota_j = lax.broadcasted_iota(jnp.int32, (n, n), 1)

    for b in range(_BM):
        rho = pk_ref[b, :, 0:n]
        theta = pk_ref[b, :, 128:128 + n]
        adj = adj_ref[b]
        ti = ti_ref[b]

        ws = []
        for i in range(nk):
            d = (rho - pp_ref[i, 0]) ** 2
            w_r = jnp.exp(-0.5 * d / (1e-14 + pp_ref[i, 2] ** 2))
            fa = jnp.abs(theta - pp_ref[i, 1])
            sa = jnp.abs(2.0 * math.pi - fa)
            ang = jnp.minimum(fa, sa)
            w_t = jnp.exp(-0.5 * ang * ang / (1e-14 + pp_ref[i, 3] ** 2))
            w = w_r * w_t
            w = jnp.where(jnp.isnan(w), 0.0, w)
            ws.append(w)
        wsum = ws[0]
        for i in range(1, nk):
            wsum = wsum + ws[i]
        inv = 1.0 / (wsum + 1e-14)

        acc = jnp.zeros((n, n), dtype=jnp.float32)
        for kk in range(k):
            idx = ti[:, kk:kk + 1]
            val = adj[:, kk:kk + 1]
            acc = acc + jnp.where(iota_j == idx, val, 0.0)
        scaled = acc * inv

        base = b * n
        wv_b = wv_ref[base:base + n, :]
        for i in range(nk):
            mi = ws[i] * scaled
            out_ref[b, :, i * 128:(i + 1) * 128] = jnp.dot(
                mi, wv_b[:, i * 128:(i + 1) * 128],
                preferred_element_type=jnp.float32)


@jax.jit
def _graph_conv(v, pk, adj, ti, W, pp):
    grid = _B // _BM
    out = pl.pallas_call(
        _graph_conv_body,
        grid=(grid,),
        in_specs=[
            pl.BlockSpec((_BM, _N, _PK), lambda i: (i, 0, 0)),
            pl.BlockSpec((_BM, _N, _K), lambda i: (i, 0, 0)),
            pl.BlockSpec((_BM, _N, _K), lambda i: (i, 0, 0)),
            pl.BlockSpec((_BM, _N, _FEAT), lambda i: (i, 0, 0)),
            pl.BlockSpec((_NK, 128, _FEAT), lambda i: (0, 0, 0)),
            pl.BlockSpec((_NK, 4), lambda i: (0, 0)),
        ],
        out_specs=pl.BlockSpec((_BM, _N, _MID), lambda i: (i, 0, 0)),
        out_shape=jax.ShapeDtypeStruct((_B, _N, _MID), jnp.float32),
        scratch_shapes=[pltpu.VMEM((_BMN, _MID), jnp.float32)],
    )(pk, adj, ti, v, W, pp)
    return out


def kernel(v, v_mask, coord, adj_matrix, top_ind, W, mean_rho, mean_theta,
           precision_rho, precision_theta):
    del v_mask
    z92 = jnp.zeros((_B, _N, 128 - _N), jnp.float32)
    pk = jnp.concatenate(
        [coord[:, :, :, 0], z92, coord[:, :, :, 1], z92], axis=-1)
    pp = jnp.concatenate(
        [mean_rho, mean_theta, precision_rho, precision_theta], axis=-1)
    return _graph_conv(v, pk, adj_matrix,
                       top_ind.astype(jnp.int32), W, pp)
